# Initial kernel scaffold; baseline (speedup 1.0000x reference)
import jax
import jax.numpy as jnp
from jax.experimental import pallas as pl


def kernel(x, edge_index_rel0, edge_index_rel1, edge_index_rel2, W0, W1, W2):
    raise NotImplementedError("write your pallas kernel here")


# speedup vs baseline: 17.2992x; 17.2992x over previous
import functools

import jax
import jax.numpy as jnp
from jax import lax
from jax.experimental import pallas as pl
from jax.experimental.pallas import tpu as pltpu
from jax.experimental.pallas import tpu_sc as plsc

N = 10000
D = 128
E = 320000
NC, NS = 2, 16
NW = NC * NS
K = 128
J = 8
N_PAD = 10240
E_PAD = 327680
ROWS = E_PAD // K
ROWS_PER_TILE = ROWS // NW
BLKS = ROWS_PER_TILE // J
NPT = N_PAD // NS
BN = 512


def _mesh():
    return plsc.VectorSubcoreMesh(core_axis_name="c", subcore_axis_name="s")


def _hist_body(i0, i1, i2, i3, i4, i5, o0, o1, o2, o3, o4, o5,
               idx_v, ones_v, zb_v,
               h0, h1, h2, h3, h4, h5):
    c = lax.axis_index("c")
    s = lax.axis_index("s")
    ins = (i0, i1, i2, i3, i4, i5)
    outs = (o0, o1, o2, o3, o4, o5)
    hsps = (h0, h1, h2, h3, h4, h5)

    def fill_ones(i, carry):
        ones_v[pl.ds(i * 16, 16)] = jnp.ones((16,), jnp.float32)
        return carry

    lax.fori_loop(0, K // 16, fill_ones, 0)

    def fill_zeros(i, carry):
        zb_v[pl.ds(i * 16, 16)] = jnp.zeros((16,), jnp.float32)
        return carry

    lax.fori_loop(0, NPT // 16, fill_zeros, 0)

    for hsp in hsps:
        pltpu.sync_copy(zb_v, hsp.at[pl.ds(s * NPT, NPT)])
    plsc.subcore_barrier()

    base_row = (c * NS + s) * ROWS_PER_TILE
    for ihbm, hsp in zip(ins, hsps):
        def blk_body(b, carry):
            pltpu.sync_copy(ihbm.at[pl.ds(base_row + b * J, J)], idx_v)
            for j in range(J):
                pltpu.sync_copy(ones_v, hsp.at[idx_v.at[j]], add=True)
            return carry

        lax.fori_loop(0, BLKS, blk_body, 0)
    plsc.subcore_barrier()

    for hsp, ohbm in zip(hsps, outs):
        pltpu.sync_copy(hsp.at[pl.ds(s * NPT, NPT)],
                        ohbm.at[pl.ds(c * N_PAD + s * NPT, NPT)])


def _hist_call(idx_arrays):
    out_t = tuple(jax.ShapeDtypeStruct((NC * N_PAD,), jnp.float32)
                  for _ in range(6))
    k = pl.kernel(
        _hist_body,
        out_type=out_t,
        mesh=_mesh(),
        scratch_types=[
            pltpu.VMEM((J, K), jnp.int32),
            pltpu.VMEM((K,), jnp.float32),
            pltpu.VMEM((NPT,), jnp.float32),
        ] + [pltpu.VMEM_SHARED((N_PAD,), jnp.float32) for _ in range(6)],
    )
    return k(*idx_arrays)


def _prescale_body(hist_ref, x_ref, xs0_ref, xs1_ref, xs2_ref):
    hist = hist_ref[...]
    deg = hist[0] + hist[1]
    x = x_ref[...]
    for r, oref in enumerate((xs0_ref, xs1_ref, xs2_ref)):
        nrm = lax.rsqrt(jnp.maximum(deg[2 * r], 1.0))
        oref[...] = x * nrm[:, None]


def _prescale_call(hist, x_pad):
    grid = (N_PAD // BN,)
    out_t = tuple(jax.ShapeDtypeStruct((N_PAD, D), jnp.float32)
                  for _ in range(3))
    return pl.pallas_call(
        _prescale_body,
        grid=grid,
        in_specs=[
            pl.BlockSpec((NC, 6, BN), lambda i: (0, 0, i)),
            pl.BlockSpec((BN, D), lambda i: (i, 0)),
        ],
        out_specs=tuple(pl.BlockSpec((BN, D), lambda i: (i, 0))
                        for _ in range(3)),
        out_shape=out_t,
    )(hist, x_pad)


def _agg_body(xs0, xs1, xs2, s0, d0, s1, d1, s2, d2, p0, p1, p2,
              si_v, di_v, rows_v, zb_v, agg):
    c = lax.axis_index("c")
    s = lax.axis_index("s")

    def fill_zeros(i, carry):
        r = i // (D // 16)
        q = i % (D // 16)
        zb_v[r, pl.ds(q * 16, 16)] = jnp.zeros((16,), jnp.float32)
        return carry

    lax.fori_loop(0, (ROWS_PER_TILE * D) // 16, fill_zeros, 0)

    base_row = (c * NS + s) * ROWS_PER_TILE
    for xsr, srh, dsh, prh in ((xs0, s0, d0, p0),
                               (xs1, s1, d1, p1),
                               (xs2, s2, d2, p2)):
        for t in range(NPT // ROWS_PER_TILE):
            pltpu.sync_copy(
                zb_v,
                agg.at[pl.ds(s * NPT + t * ROWS_PER_TILE,
                             ROWS_PER_TILE), :])
        plsc.subcore_barrier()

        def blk_body(b, carry):
            pltpu.sync_copy(srh.at[pl.ds(base_row + b * J, J)], si_v)
            pltpu.sync_copy(dsh.at[pl.ds(base_row + b * J, J)], di_v)
            for j in range(J):
                pltpu.sync_copy(xsr.at[si_v.at[j]], rows_v)
                pltpu.sync_copy(rows_v, agg.at[di_v.at[j]], add=True)
            return carry

        lax.fori_loop(0, BLKS, blk_body, 0)
        plsc.subcore_barrier()

        pltpu.sync_copy(agg.at[pl.ds(s * NPT, NPT), :],
                        prh.at[pl.ds(c * N_PAD + s * NPT, NPT), :])
        plsc.subcore_barrier()


def _agg_call(xs, idx_arrays):
    out_t = tuple(jax.ShapeDtypeStruct((NC * N_PAD, D), jnp.float32)
                  for _ in range(3))
    k = pl.kernel(
        _agg_body,
        out_type=out_t,
        mesh=_mesh(),
        scratch_types=[
            pltpu.VMEM((J, K), jnp.int32),
            pltpu.VMEM((J, K), jnp.int32),
            pltpu.VMEM((K, D), jnp.float32),
            pltpu.VMEM((ROWS_PER_TILE, D), jnp.float32),
            pltpu.VMEM_SHARED((N_PAD, D), jnp.float32),
        ],
    )
    return k(xs[0], xs[1], xs[2], *idx_arrays)


def _final_body(hist_ref, p0_ref, p1_ref, p2_ref, w_ref, o_ref):
    hist = hist_ref[...]
    acc = jnp.zeros((BN, D), jnp.float32)
    for r, pref in enumerate((p0_ref, p1_ref, p2_ref)):
        p = pref[...]
        deg_in = hist[0, 2 * r + 1] + hist[1, 2 * r + 1]
        nin = lax.rsqrt(jnp.maximum(deg_in, 1.0))
        m = (p[0] + p[1]) * nin[:, None]
        acc = acc + jnp.dot(m, w_ref[r],
                            preferred_element_type=jnp.float32)
    o_ref[...] = acc


def _final_call(hist, parts, w_stack):
    grid = (N_PAD // BN,)
    return pl.pallas_call(
        _final_body,
        grid=grid,
        in_specs=[
            pl.BlockSpec((NC, 6, BN), lambda i: (0, 0, i)),
            pl.BlockSpec((NC, BN, D), lambda i: (0, i, 0)),
            pl.BlockSpec((NC, BN, D), lambda i: (0, i, 0)),
            pl.BlockSpec((NC, BN, D), lambda i: (0, i, 0)),
            pl.BlockSpec((3, D, D), lambda i: (0, 0, 0)),
        ],
        out_specs=pl.BlockSpec((BN, D), lambda i: (i, 0)),
        out_shape=jax.ShapeDtypeStruct((N_PAD, D), jnp.float32),
    )(hist, parts[0], parts[1], parts[2], w_stack)


def _pad_idx(a):
    pad = (jnp.arange(E_PAD - E, dtype=jnp.int32) % (N_PAD - N)) + N
    return jnp.concatenate([a, pad]).reshape(ROWS, K)


def kernel(x, edge_index_rel0, edge_index_rel1, edge_index_rel2, W0, W1, W2):
    idx = []
    for ei in (edge_index_rel0, edge_index_rel1, edge_index_rel2):
        idx.append(_pad_idx(ei[0]))
        idx.append(_pad_idx(ei[1]))
    idx = tuple(idx)

    hists = _hist_call(idx)
    hist = jnp.stack([h.reshape(NC, N_PAD) for h in hists], axis=1)

    x_pad = jnp.pad(x, ((0, N_PAD - N), (0, 0)))
    xs = _prescale_call(hist, x_pad)

    parts = _agg_call(xs, idx)
    parts = tuple(p.reshape(NC, N_PAD, D) for p in parts)

    w_stack = jnp.stack([W0, W1, W2], axis=0)
    out = _final_call(hist, parts, w_stack)
    return out[:N]
